# baseline (speedup 1.0000x reference)
import functools

import jax
import jax.numpy as jnp
from jax import lax
from jax.experimental import pallas as pl
from jax.experimental.pallas import tpu as pltpu
from jax.experimental.pallas import tpu_sc as plsc

N = 10000
E = 320000
D_IN = 128
D_HID = 16
D_OUT = 8

NC = 2
NS = 16
NW = NC * NS
CH = 128
K = 2 * (-(-E // (NW * CH * 2)))
TCH = NW * K
E_PAD = CH * TCH
N_PAD = 10240
RPT = N_PAD // NS

DEG_KS = (88, 72)
MP16_KS = (120, 40)
MP8_KS = (88, 72)
NSLOT = 8

_mesh = plsc.VectorSubcoreMesh(
    core_axis_name="c", subcore_axis_name="s", num_cores=NC, num_subcores=NS
)


@functools.partial(
    pl.kernel,
    out_type=jax.ShapeDtypeStruct((NC, N_PAD), jnp.float32),
    mesh=_mesh,
    scratch_types=[
        pltpu.VMEM((max(DEG_KS), CH), jnp.int32),
        pltpu.VMEM((CH,), jnp.float32),
        pltpu.VMEM((RPT,), jnp.float32),
        pltpu.VMEM_SHARED((N_PAD,), jnp.float32),
    ],
    compiler_params=pltpu.CompilerParams(use_tc_tiling_on_sc=False),
)
def _deg(dst_hbm, out_hbm, dst_v, ones_v, z_v, deg_sh):
    c = lax.axis_index("c")
    s = lax.axis_index("s")
    for i in range(CH // 16):
        ones_v[pl.ds(i * 16, 16)] = jnp.ones((16,), jnp.float32)

    def zbody(i, _):
        z_v[pl.ds(i * 16, 16)] = jnp.zeros((16,), jnp.float32)
        return 0

    lax.fori_loop(0, RPT // 16, zbody, 0)
    pltpu.sync_copy(z_v, deg_sh.at[pl.ds(s * RPT, RPT)])

    def run(kc, base):
        pltpu.sync_copy(dst_hbm.at[pl.ds(base, kc)], dst_v.at[pl.ds(0, kc)])
        plsc.subcore_barrier()

        def body(j, _):
            pltpu.sync_copy(ones_v, deg_sh.at[dst_v.at[j]], add=True)
            return 0

        lax.fori_loop(0, kc, body, 0)
        plsc.subcore_barrier()

    @pl.when(c == 0)
    def _():
        run(DEG_KS[0], s * DEG_KS[0])

    @pl.when(c == 1)
    def _():
        run(DEG_KS[1], NS * DEG_KS[0] + s * DEG_KS[1])

    pltpu.sync_copy(deg_sh.at[pl.ds(s * RPT, RPT)],
                    out_hbm.at[c, pl.ds(s * RPT, RPT)])


def _make_mp(D, ks, tec_zero):
    k0, k1 = ks
    assert k0 % NSLOT == 0 and k1 % NSLOT == 0
    scratch = (
        [pltpu.VMEM((max(ks), CH), jnp.int32),
         pltpu.VMEM((max(ks), CH), jnp.int32)]
        + [pltpu.VMEM((CH, D), jnp.float32) for _ in range(NSLOT)]
        + [pltpu.VMEM_SHARED((N_PAD, D), jnp.float32)]
        + [pltpu.SemaphoreType.DMA for _ in range(2 * NSLOT)]
    )
    if tec_zero:
        scratch.append(pltpu.VMEM((RPT, D), jnp.float32))

    @functools.partial(
        pl.kernel,
        out_type=jax.ShapeDtypeStruct((NC, N_PAD, D), jnp.float32),
        mesh=_mesh,
        scratch_types=scratch,
        compiler_params=pltpu.CompilerParams(use_tc_tiling_on_sc=False),
    )
    def _mp(y_hbm, src_hbm, dst_hbm, zeros_hbm, out_hbm, src_v, dst_v,
            *rest):
        bufs = rest[:NSLOT]
        acc_sh = rest[NSLOT]
        gsem = rest[NSLOT + 1:2 * NSLOT + 1]
        ssem = rest[2 * NSLOT + 1:3 * NSLOT + 1]
        c = lax.axis_index("c")
        s = lax.axis_index("s")

        def run(kc, base):
            pltpu.sync_copy(src_hbm.at[pl.ds(base, kc)],
                            src_v.at[pl.ds(0, kc)])
            pltpu.sync_copy(dst_hbm.at[pl.ds(base, kc)],
                            dst_v.at[pl.ds(0, kc)])
            for u in range(4):
                pltpu.async_copy(y_hbm.at[src_v.at[u]], bufs[u], gsem[u])
            if tec_zero:
                zbuf = rest[3 * NSLOT + 1]

                def zb(i, _):
                    for u in range(8):
                        zbuf[i * 8 + u, :] = jnp.zeros((16,), jnp.float32)
                    return 0

                lax.fori_loop(0, RPT // 8, zb, 0)
                pltpu.sync_copy(zbuf, acc_sh.at[pl.ds(s * RPT, RPT)])
            else:
                pltpu.sync_copy(zeros_hbm.at[pl.ds(s * RPT, RPT)],
                                acc_sh.at[pl.ds(s * RPT, RPT)])
            plsc.subcore_barrier()

            def body(g, _):
                for u in range(NSLOT):
                    j = NSLOT * g + u
                    nslot = (u + 4) % NSLOT
                    pltpu.make_async_copy(y_hbm.at[src_v.at[j]], bufs[u],
                                          gsem[u]).wait()
                    pltpu.async_copy(bufs[u], acc_sh.at[dst_v.at[j]],
                                     ssem[u], add=True)

                    @pl.when(j >= 4)
                    def _():
                        pltpu.make_async_copy(
                            bufs[nslot], acc_sh.at[dst_v.at[0]],
                            ssem[nslot]).wait()
                    nxt = lax.rem(j + 4, kc)
                    pltpu.async_copy(y_hbm.at[src_v.at[nxt]], bufs[nslot],
                                     gsem[nslot])
                return 0

            lax.fori_loop(0, kc // NSLOT, body, 0)
            for u in range(4, NSLOT):
                pltpu.make_async_copy(bufs[u], acc_sh.at[dst_v.at[0]],
                                      ssem[u]).wait()
            for u in range(4):
                pltpu.make_async_copy(y_hbm.at[src_v.at[0]], bufs[u],
                                      gsem[u]).wait()
            plsc.subcore_barrier()

        @pl.when(c == 0)
        def _():
            run(k0, s * k0)

        @pl.when(c == 1)
        def _():
            run(k1, NS * k0 + s * k1)

        pltpu.sync_copy(acc_sh.at[pl.ds(s * RPT, RPT)],
                        out_hbm.at[c, pl.ds(s * RPT, RPT)])

    return _mp


_mp16 = _make_mp(D_HID, MP16_KS, tec_zero=True)
_mp8 = _make_mp(D_OUT, MP8_KS, tec_zero=False)


def _d1_body(x_ref, w1_ref, degp_ref, y1_ref, dinv_ref):
    deg = degp_ref[0] + degp_ref[1] + 1.0
    dinv = lax.rsqrt(deg)
    dinv_ref[...] = dinv
    y1_ref[...] = jnp.dot(x_ref[...], w1_ref[...],
                          preferred_element_type=jnp.float32) * dinv


def _d2_body(p_ref, y1_ref, dinv_ref, b1_ref, w2_ref, y2_ref):
    dinv = dinv_ref[...]
    agg = (p_ref[0] + p_ref[1] + y1_ref[...]) * dinv + b1_ref[...]
    h = jnp.maximum(agg, 0.0)
    y2_ref[...] = jnp.dot(h, w2_ref[...],
                          preferred_element_type=jnp.float32) * dinv


def _d3_body(p_ref, y2_ref, dinv_ref, b2_ref, out_ref):
    z = p_ref[0, :N] + p_ref[1, :N] + y2_ref[:N]
    z = z * dinv_ref[:N] + b2_ref[...]
    m = jnp.max(z, axis=1, keepdims=True)
    e = jnp.exp(z - m)
    out_ref[...] = e / jnp.sum(e, axis=1, keepdims=True)


def kernel(x, edge_index, W1, b1, W2, b2):
    src = edge_index[0]
    dst = edge_index[1]
    fill = jnp.full((TCH - E // CH, CH), N, jnp.int32)
    srcp = jnp.concatenate([src.reshape(E // CH, CH), fill])
    dstp = jnp.concatenate([dst.reshape(E // CH, CH), fill])
    xp = jnp.pad(x, ((0, N_PAD - N), (0, 0)))
    z16 = jnp.zeros((N_PAD, D_HID), jnp.float32)
    z8 = jnp.zeros((N_PAD, D_OUT), jnp.float32)

    degp = _deg(dstp).reshape(NC, N_PAD, 1)

    y1, dinv = pl.pallas_call(
        _d1_body,
        out_shape=(
            jax.ShapeDtypeStruct((N_PAD, D_HID), jnp.float32),
            jax.ShapeDtypeStruct((N_PAD, 1), jnp.float32),
        ),
    )(xp, W1, degp)

    p1 = _mp16(y1, srcp, dstp, z16)

    y2 = pl.pallas_call(
        _d2_body,
        out_shape=jax.ShapeDtypeStruct((N_PAD, D_OUT), jnp.float32),
    )(p1, y1, dinv, b1.reshape(1, D_HID), W2)

    p2 = _mp8(y2, srcp, dstp, z8)

    out = pl.pallas_call(
        _d3_body,
        out_shape=jax.ShapeDtypeStruct((N, D_OUT), jnp.float32),
    )(p2, y2, dinv, b2.reshape(1, D_OUT))
    return out


# speedup vs baseline: 1.0169x; 1.0169x over previous
import functools

import jax
import jax.numpy as jnp
from jax import lax
from jax.experimental import pallas as pl
from jax.experimental.pallas import tpu as pltpu
from jax.experimental.pallas import tpu_sc as plsc

N = 10000
E = 320000
D_IN = 128
D_HID = 16
D_OUT = 8

NC = 2
NS = 16
NW = NC * NS
CH = 128
K = 2 * (-(-E // (NW * CH * 2)))
TCH = NW * K
E_PAD = CH * TCH
N_PAD = 10240
RPT = N_PAD // NS

DEG_KS = (88, 72)
MP16_KS = (118, 42)
MP8_KS = (88, 72)

_mesh = plsc.VectorSubcoreMesh(
    core_axis_name="c", subcore_axis_name="s", num_cores=NC, num_subcores=NS
)


@functools.partial(
    pl.kernel,
    out_type=jax.ShapeDtypeStruct((NC, N_PAD), jnp.float32),
    mesh=_mesh,
    scratch_types=[
        pltpu.VMEM((max(DEG_KS), CH), jnp.int32),
        pltpu.VMEM((CH,), jnp.float32),
        pltpu.VMEM((RPT,), jnp.float32),
        pltpu.VMEM_SHARED((N_PAD,), jnp.float32),
    ],
    compiler_params=pltpu.CompilerParams(use_tc_tiling_on_sc=False),
)
def _deg(dst_hbm, out_hbm, dst_v, ones_v, z_v, deg_sh):
    c = lax.axis_index("c")
    s = lax.axis_index("s")
    for i in range(CH // 16):
        ones_v[pl.ds(i * 16, 16)] = jnp.ones((16,), jnp.float32)

    def zbody(i, _):
        z_v[pl.ds(i * 16, 16)] = jnp.zeros((16,), jnp.float32)
        return 0

    lax.fori_loop(0, RPT // 16, zbody, 0)
    pltpu.sync_copy(z_v, deg_sh.at[pl.ds(s * RPT, RPT)])

    def run(kc, base):
        pltpu.sync_copy(dst_hbm.at[pl.ds(base, kc)], dst_v.at[pl.ds(0, kc)])
        plsc.subcore_barrier()

        def body(j, _):
            pltpu.sync_copy(ones_v, deg_sh.at[dst_v.at[j]], add=True)
            return 0

        lax.fori_loop(0, kc, body, 0)
        plsc.subcore_barrier()

    @pl.when(c == 0)
    def _():
        run(DEG_KS[0], s * DEG_KS[0])

    @pl.when(c == 1)
    def _():
        run(DEG_KS[1], NS * DEG_KS[0] + s * DEG_KS[1])

    pltpu.sync_copy(deg_sh.at[pl.ds(s * RPT, RPT)],
                    out_hbm.at[c, pl.ds(s * RPT, RPT)])


def _make_mp(D, ks, tec_zero):
    k0, k1 = ks
    scratch = [
        pltpu.VMEM((max(ks), CH), jnp.int32),
        pltpu.VMEM((max(ks), CH), jnp.int32),
        pltpu.VMEM((CH, D), jnp.float32),
        pltpu.VMEM((CH, D), jnp.float32),
        pltpu.VMEM_SHARED((N_PAD, D), jnp.float32),
        pltpu.SemaphoreType.DMA,
        pltpu.SemaphoreType.DMA,
    ]
    if tec_zero:
        scratch.append(pltpu.VMEM((RPT, D), jnp.float32))

    @functools.partial(
        pl.kernel,
        out_type=jax.ShapeDtypeStruct((NC, N_PAD, D), jnp.float32),
        mesh=_mesh,
        scratch_types=scratch,
        compiler_params=pltpu.CompilerParams(use_tc_tiling_on_sc=False),
    )
    def _mp(y_hbm, src_hbm, dst_hbm, zeros_hbm, out_hbm, src_v, dst_v,
            buf0, buf1, acc_sh, sem0, sem1, *maybe_zbuf):
        c = lax.axis_index("c")
        s = lax.axis_index("s")

        def run(kc, base):
            pltpu.sync_copy(src_hbm.at[pl.ds(base, kc)],
                            src_v.at[pl.ds(0, kc)])
            pltpu.sync_copy(dst_hbm.at[pl.ds(base, kc)],
                            dst_v.at[pl.ds(0, kc)])
            pltpu.async_copy(y_hbm.at[src_v.at[0]], buf0, sem0)
            if tec_zero:
                zbuf = maybe_zbuf[0]

                def zb(i, _):
                    for u in range(8):
                        zbuf[i * 8 + u, :] = jnp.zeros((16,), jnp.float32)
                    return 0

                lax.fori_loop(0, RPT // 8, zb, 0)
                pltpu.sync_copy(zbuf, acc_sh.at[pl.ds(s * RPT, RPT)])
            else:
                pltpu.sync_copy(zeros_hbm.at[pl.ds(s * RPT, RPT)],
                                acc_sh.at[pl.ds(s * RPT, RPT)])
            plsc.subcore_barrier()

            def body(g, _):
                a = 2 * g
                b = a + 1
                nxt = lax.rem(a + 2, kc)
                pltpu.async_copy(y_hbm.at[src_v.at[b]], buf1, sem1)
                pltpu.make_async_copy(y_hbm.at[src_v.at[a]], buf0,
                                      sem0).wait()
                pltpu.sync_copy(buf0, acc_sh.at[dst_v.at[a]], add=True)
                pltpu.async_copy(y_hbm.at[src_v.at[nxt]], buf0, sem0)
                pltpu.make_async_copy(y_hbm.at[src_v.at[b]], buf1,
                                      sem1).wait()
                pltpu.sync_copy(buf1, acc_sh.at[dst_v.at[b]], add=True)
                return 0

            lax.fori_loop(0, kc // 2, body, 0)
            pltpu.make_async_copy(y_hbm.at[src_v.at[0]], buf0, sem0).wait()
            plsc.subcore_barrier()

        @pl.when(c == 0)
        def _():
            run(k0, s * k0)

        @pl.when(c == 1)
        def _():
            run(k1, NS * k0 + s * k1)

        pltpu.sync_copy(acc_sh.at[pl.ds(s * RPT, RPT)],
                        out_hbm.at[c, pl.ds(s * RPT, RPT)])

    return _mp


_mp16 = _make_mp(D_HID, MP16_KS, tec_zero=True)
_mp8 = _make_mp(D_OUT, MP8_KS, tec_zero=False)


def _d1_body(x_ref, w1_ref, degp_ref, y1_ref, dinv_ref):
    deg = degp_ref[0] + degp_ref[1] + 1.0
    dinv = lax.rsqrt(deg)
    dinv_ref[...] = dinv
    y1_ref[...] = jnp.dot(x_ref[...], w1_ref[...],
                          preferred_element_type=jnp.float32) * dinv


def _d2_body(p_ref, y1_ref, dinv_ref, b1_ref, w2_ref, y2_ref):
    dinv = dinv_ref[...]
    agg = (p_ref[0] + p_ref[1] + y1_ref[...]) * dinv + b1_ref[...]
    h = jnp.maximum(agg, 0.0)
    y2_ref[...] = jnp.dot(h, w2_ref[...],
                          preferred_element_type=jnp.float32) * dinv


def _d3_body(p_ref, y2_ref, dinv_ref, b2_ref, out_ref):
    z = p_ref[0, :N] + p_ref[1, :N] + y2_ref[:N]
    z = z * dinv_ref[:N] + b2_ref[...]
    m = jnp.max(z, axis=1, keepdims=True)
    e = jnp.exp(z - m)
    out_ref[...] = e / jnp.sum(e, axis=1, keepdims=True)


def kernel(x, edge_index, W1, b1, W2, b2):
    src = edge_index[0]
    dst = edge_index[1]
    fill = jnp.full((TCH - E // CH, CH), N, jnp.int32)
    srcp = jnp.concatenate([src.reshape(E // CH, CH), fill])
    dstp = jnp.concatenate([dst.reshape(E // CH, CH), fill])
    xp = jnp.pad(x, ((0, N_PAD - N), (0, 0)))
    z16 = jnp.zeros((N_PAD, D_HID), jnp.float32)
    z8 = jnp.zeros((N_PAD, D_OUT), jnp.float32)

    degp = _deg(dstp).reshape(NC, N_PAD, 1)

    y1, dinv = pl.pallas_call(
        _d1_body,
        out_shape=(
            jax.ShapeDtypeStruct((N_PAD, D_HID), jnp.float32),
            jax.ShapeDtypeStruct((N_PAD, 1), jnp.float32),
        ),
    )(xp, W1, degp)

    p1 = _mp16(y1, srcp, dstp, z16)

    y2 = pl.pallas_call(
        _d2_body,
        out_shape=jax.ShapeDtypeStruct((N_PAD, D_OUT), jnp.float32),
    )(p1, y1, dinv, b1.reshape(1, D_HID), W2)

    p2 = _mp8(y2, srcp, dstp, z8)

    out = pl.pallas_call(
        _d3_body,
        out_shape=jax.ShapeDtypeStruct((N, D_OUT), jnp.float32),
    )(p2, y2, dinv, b2.reshape(1, D_OUT))
    return out
